# Initial kernel scaffold; baseline (speedup 1.0000x reference)
import jax
import jax.numpy as jnp
from jax.experimental import pallas as pl


def kernel(text, image, audio, modality_embed):
    raise NotImplementedError("write your pallas kernel here")


# speedup vs baseline: 1.9973x; 1.9973x over previous
import jax
import jax.numpy as jnp
from jax import lax
from jax.experimental import pallas as pl

_C = 512


def _body(t_ref, i_ref, a_ref, emb_ref, out_ref, ids_ref, *, n_t, n_i, l_t, l_i, tot):
    b = pl.program_id(0)
    j = pl.program_id(1)

    @pl.when(j < n_t)
    def _():
        out_ref[...] = t_ref[...] + emb_ref[0, :][None, None, :]

    @pl.when((j >= n_t) & (j < n_t + n_i))
    def _():
        out_ref[...] = i_ref[...] + emb_ref[1, :][None, None, :]

    @pl.when(j >= n_t + n_i)
    def _():
        out_ref[...] = a_ref[...] + emb_ref[2, :][None, None, :]

    @pl.when((b == 0) & (j == 0))
    def _():
        col = lax.broadcasted_iota(jnp.int32, ids_ref.shape, 1)
        ids_ref[...] = (col >= l_t).astype(jnp.int32) + (col >= l_t + l_i).astype(jnp.int32)


def kernel(text, image, audio, modality_embed):
    B, l_t, H = text.shape
    l_i = image.shape[1]
    l_a = audio.shape[1]
    tot = l_t + l_i + l_a
    n_t, n_i, n_a = l_t // _C, l_i // _C, l_a // _C
    nc = n_t + n_i + n_a

    import functools
    body = functools.partial(_body, n_t=n_t, n_i=n_i, l_t=l_t, l_i=l_i, tot=tot)

    out, ids = pl.pallas_call(
        body,
        grid=(B, nc),
        in_specs=[
            pl.BlockSpec((1, _C, H), lambda b, j: (b, jnp.minimum(j, n_t - 1), 0)),
            pl.BlockSpec((1, _C, H), lambda b, j: (b, jnp.clip(j - n_t, 0, n_i - 1), 0)),
            pl.BlockSpec((1, _C, H), lambda b, j: (b, jnp.clip(j - n_t - n_i, 0, n_a - 1), 0)),
            pl.BlockSpec(modality_embed.shape, lambda b, j: (0, 0)),
        ],
        out_specs=[
            pl.BlockSpec((1, _C, H), lambda b, j: (b, j, 0)),
            pl.BlockSpec((B, tot), lambda b, j: (0, 0)),
        ],
        out_shape=[
            jax.ShapeDtypeStruct((B, tot, H), jnp.float32),
            jax.ShapeDtypeStruct((B, tot), jnp.int32),
        ],
    )(text, image, audio, modality_embed)
    return out, ids
